# baseline (speedup 1.0000x reference)
import functools

import jax
import jax.numpy as jnp
from jax import lax
from jax.experimental import pallas as pl
from jax.experimental.pallas import tpu as pltpu
from jax.experimental.pallas import tpu_sc as plsc

_K = 256.0
_B = 64
_T = 32768
_L = 16

_NC, _NS = 2, 16

_SC_ROWS = 8
_RPC = _SC_ROWS // _NC
_WPR = _NS // _RPC
_SEG = _T // _WPR
_NCH = _SEG // _L
_TC_ROWS = _B - _SC_ROWS


def _sigmoid(x, it):
    return 1.0 / (1.0 + jnp.exp(-(x * it)))


def _sc_body(scores_hbm, invt_hbm, out_hbm, x_v, y_v, o_v, it_v, tmp_v,
             shared, sem):
    c = lax.axis_index("c")
    s = lax.axis_index("s")
    row_local = s // _WPR
    seg = s % _WPR
    r_out = c * _RPC + row_local
    r_in = _TC_ROWS + r_out
    start = seg * _SEG
    hstart = (start + _SEG) & (_T - 1)

    pltpu.sync_copy(invt_hbm, it_v)
    it = it_v[...]

    pltpu.sync_copy(scores_hbm.at[pl.ds(r_in * _T + start, _SEG)],
                    x_v.at[pl.ds(0, _SEG)])
    h_off = pl.multiple_of(r_in * _T + hstart, 8)
    pltpu.sync_copy(scores_hbm.at[pl.ds(h_off, _L)],
                    x_v.at[pl.ds(_SEG, _L)])

    def pass1(ch, acc):
        y = _sigmoid(x_v[pl.ds(ch * _L, _L)], it)
        y_v[pl.ds(ch * _L, _L)] = y
        return acc + y

    acc = lax.fori_loop(0, _NCH, pass1, jnp.zeros((_L,), jnp.float32))
    y_v[pl.ds(_SEG, _L)] = _sigmoid(x_v[pl.ds(_SEG, _L)], it)

    tmp_v[...] = acc
    pltpu.sync_copy(tmp_v, shared.at[8 + s])
    plsc.subcore_barrier()
    sum_v = jnp.zeros((_L,), jnp.float32)
    for k in range(_WPR):
        pltpu.sync_copy(shared.at[8 + row_local * _WPR + k], tmp_v)
        sum_v = sum_v + tmp_v[...]
    total = sum_v[0]
    for j in range(1, _L):
        total = total + sum_v[j]
    total_v = jnp.broadcast_to(total, (_L,))
    scale = jnp.minimum(_K / jnp.maximum(total_v, 1e-6), 1.0)

    def pass2(ch, carry):
        base = ch * _L
        y0 = y_v[pl.ds(base, _L)] * scale
        y1 = y_v[pl.ds(base + 1, _L)] * scale
        y2 = y_v[pl.ds(base + 2, _L)] * scale
        y3 = y_v[pl.ds(base + 3, _L)] * scale
        z0 = y0 * jnp.minimum(2.0 / (1.0 + y0 + y1), 1.0)
        z2 = y2 * jnp.minimum(2.0 / (1.0 + y2 + y3), 1.0)
        w = z0 * jnp.minimum(2.0 / (1.0 + z0 + z2), 1.0)
        o_v[pl.ds(base, _L)] = w
        return carry

    lax.fori_loop(0, _NCH, pass2, jnp.int32(0))

    @pl.when(seg == 0)
    def _():
        lane = lax.iota(jnp.int32, _L)
        o_v[pl.ds(0, _L)] = jnp.where(lane == 0, 0.0, o_v[pl.ds(0, _L)])

    pltpu.sync_copy(o_v, out_hbm.at[pl.ds(r_out * _T + start, _SEG)])


def _sc_call(scores, inv_temp_vec):
    f = functools.partial(
        pl.kernel,
        mesh=plsc.VectorSubcoreMesh(core_axis_name="c", subcore_axis_name="s"),
        out_type=jax.ShapeDtypeStruct((_SC_ROWS * _T,), jnp.float32),
        scratch_types=[
            pltpu.VMEM((_SEG + _L,), jnp.float32),
            pltpu.VMEM((_SEG + _L,), jnp.float32),
            pltpu.VMEM((_SEG,), jnp.float32),
            pltpu.VMEM((_L,), jnp.float32),
            pltpu.VMEM((_L,), jnp.float32),
            pltpu.VMEM_SHARED((_NS + 8, _L), jnp.float32),
            pltpu.SemaphoreType.DMA,
        ],
    )(_sc_body)
    return f(scores, inv_temp_vec)


_TC_BLOCK = 8


def _tc_body(scale_ref, x_ref, o_ref):
    inv_temp = scale_ref[0]
    y = jax.nn.sigmoid(x_ref[...] * inv_temp)
    budget = jnp.clip(jnp.sum(y, axis=1, keepdims=True), 1e-6, None)
    y = y * jnp.minimum(_K / budget, 1.0)
    for d in (1, 2):
        shifted = pltpu.roll(y, shift=_T - d, axis=1)
        y = y * jnp.minimum(2.0 / (1.0 + y + shifted), 1.0)
    col = jax.lax.broadcasted_iota(jnp.int32, y.shape, 1)
    o_ref[...] = jnp.where(col == 0, 0.0, y)


def _tc_call(scores, inv_temp_smem):
    return pl.pallas_call(
        _tc_body,
        grid=(_TC_ROWS // _TC_BLOCK,),
        in_specs=[
            pl.BlockSpec(memory_space=pltpu.SMEM),
            pl.BlockSpec((_TC_BLOCK, _T), lambda i: (i, 0)),
        ],
        out_specs=pl.BlockSpec((_TC_BLOCK, _T), lambda i: (i, 0)),
        out_shape=jax.ShapeDtypeStruct((_B, _T), jnp.float32),
        compiler_params=pltpu.CompilerParams(
            dimension_semantics=("arbitrary",),
        ),
    )(inv_temp_smem, scores)


@jax.jit
def kernel(scores, log_temperature):
    temp = jnp.clip(jnp.exp(log_temperature), 0.1, 10.0)
    inv_temp = (1.0 / temp).astype(jnp.float32)
    sc_out = _sc_call(scores.reshape(-1), jnp.broadcast_to(inv_temp, (_L,)))
    tc_out = _tc_call(scores, inv_temp.reshape(1))
    return lax.dynamic_update_slice(
        tc_out, sc_out.reshape(_SC_ROWS, _T), (_TC_ROWS, 0))


# speedup vs baseline: 3.1206x; 3.1206x over previous
import functools

import jax
import jax.numpy as jnp
from jax.experimental import pallas as pl
from jax.experimental.pallas import tpu as pltpu

_K = 256.0
_B = 64
_T = 32768
_ROWS_PER_BLOCK = 8


def _tc_body(scale_ref, x_ref, o_ref):
    inv_temp = scale_ref[0]
    y = jax.nn.sigmoid(x_ref[...] * inv_temp)
    budget = jnp.clip(jnp.sum(y, axis=1, keepdims=True), 1e-6, None)
    y = y * jnp.minimum(_K / budget, 1.0)
    for d in (1, 2):
        shifted = pltpu.roll(y, shift=_T - d, axis=1)
        y = y * jnp.minimum(2.0 / (1.0 + y + shifted), 1.0)
    col = jax.lax.broadcasted_iota(jnp.int32, y.shape, 1)
    o_ref[...] = jnp.where(col == 0, 0.0, y)


@jax.jit
def kernel(scores, log_temperature):
    temp = jnp.clip(jnp.exp(log_temperature), 0.1, 10.0)
    inv_temp = (1.0 / temp).reshape(1).astype(jnp.float32)
    grid = (_B // _ROWS_PER_BLOCK,)
    return pl.pallas_call(
        _tc_body,
        grid=grid,
        in_specs=[
            pl.BlockSpec(memory_space=pltpu.SMEM),
            pl.BlockSpec((_ROWS_PER_BLOCK, _T), lambda i: (i, 0)),
        ],
        out_specs=pl.BlockSpec((_ROWS_PER_BLOCK, _T), lambda i: (i, 0)),
        out_shape=jax.ShapeDtypeStruct((_B, _T), jnp.float32),
        compiler_params=pltpu.CompilerParams(
            dimension_semantics=("parallel",),
        ),
    )(inv_temp, scores)


# speedup vs baseline: 3.5210x; 1.1283x over previous
import functools

import jax
import jax.numpy as jnp
from jax.experimental import pallas as pl
from jax.experimental.pallas import tpu as pltpu

_K = 256.0
_B = 64
_T = 32768
_ROWS_PER_BLOCK = 16


def _tc_body(scale_ref, x_ref, o_ref):
    inv_temp = scale_ref[0]
    y = jax.nn.sigmoid(x_ref[...] * inv_temp)
    budget = jnp.clip(jnp.sum(y, axis=1, keepdims=True), 1e-6, None)
    y = y * jnp.minimum(_K / budget, 1.0)
    for d in (1, 2):
        shifted = pltpu.roll(y, shift=_T - d, axis=1)
        y = y * jnp.minimum(2.0 / (1.0 + y + shifted), 1.0)
    col = jax.lax.broadcasted_iota(jnp.int32, y.shape, 1)
    o_ref[...] = jnp.where(col == 0, 0.0, y)


@jax.jit
def kernel(scores, log_temperature):
    temp = jnp.clip(jnp.exp(log_temperature), 0.1, 10.0)
    inv_temp = (1.0 / temp).reshape(1).astype(jnp.float32)
    grid = (_B // _ROWS_PER_BLOCK,)
    return pl.pallas_call(
        _tc_body,
        grid=grid,
        in_specs=[
            pl.BlockSpec(memory_space=pltpu.SMEM),
            pl.BlockSpec((_ROWS_PER_BLOCK, _T), lambda i: (i, 0)),
        ],
        out_specs=pl.BlockSpec((_ROWS_PER_BLOCK, _T), lambda i: (i, 0)),
        out_shape=jax.ShapeDtypeStruct((_B, _T), jnp.float32),
        compiler_params=pltpu.CompilerParams(
            dimension_semantics=("arbitrary",),
        ),
    )(inv_temp, scores)
